# baseline (speedup 1.0000x reference)
import functools

import jax
import jax.numpy as jnp
from jax import lax
from jax.experimental import pallas as pl
from jax.experimental.pallas import tpu as pltpu
from jax.experimental.pallas import tpu_sc as plsc

H = 64
D = 2 * H
B = 1024
L = 50
MAX_NORM = 1.5

N_WORKERS = 32
ROWS = B * L
B_PER_W = ROWS // N_WORKERS
GATHER_CHUNKS = [128] * 12 + [64]

BB = 8
R = BB * L


def _gather_rows(item_table, idx_flat):
    mesh = plsc.VectorSubcoreMesh(core_axis_name="c", subcore_axis_name="s")

    @functools.partial(
        pl.kernel,
        mesh=mesh,
        out_type=jax.ShapeDtypeStruct((ROWS, H), jnp.float32),
        scratch_types=[
            pltpu.VMEM((B_PER_W,), jnp.int32),
            pltpu.VMEM((B_PER_W, H), jnp.float32),
            pltpu.SemaphoreType.DMA,
        ],
        compiler_params=pltpu.CompilerParams(use_tc_tiling_on_sc=False),
    )
    def gather_kernel(table_hbm, idx_hbm, out_hbm, idx_v, rows_v, sem):
        wid = lax.axis_index("s") * 2 + lax.axis_index("c")
        base = wid * B_PER_W
        pltpu.sync_copy(idx_hbm.at[pl.ds(base, B_PER_W)], idx_v)
        copies = []
        off = 0
        for sz in GATHER_CHUNKS:
            copies.append(
                pltpu.async_copy(
                    table_hbm.at[idx_v.at[pl.ds(off, sz)]],
                    rows_v.at[pl.ds(off, sz)],
                    sem,
                )
            )
            off += sz
        for c in copies:
            c.wait()
        pltpu.sync_copy(rows_v, out_hbm.at[pl.ds(base, B_PER_W)])

    return gather_kernel(item_table, idx_flat)


def _renorm(e):
    n = jnp.sqrt(jnp.sum(e * e, axis=-1, keepdims=True))
    return e * jnp.minimum(1.0, MAX_NORM / jnp.maximum(n, 1e-7))


def _dense_body(rows_ref, pos_ref, a_in_ref, a_out_ref, lnw_ref, lnb_ref,
                win_ref, bin_ref, wout_ref, bout_ref, wih_ref, whh_ref,
                bih_ref, bhh_ref, biah_ref, boah_ref, out_ref):
    dot_t = lambda x, w: lax.dot_general(
        x, w, (((1,), (1,)), ((), ())), preferred_element_type=jnp.float32)

    e = _renorm(rows_ref[...])
    p = _renorm(pos_ref[...])
    p = jnp.concatenate([p] * BB, axis=0)
    seq = jnp.concatenate([e, p], axis=1)

    u = jnp.mean(seq, axis=1, keepdims=True)
    s = jnp.mean((seq - u) ** 2, axis=1, keepdims=True)
    hidden = lnw_ref[...] * ((seq - u) / jnp.sqrt(s + 1e-12)) + lnb_ref[...]

    ein = dot_t(hidden, win_ref[...]) + bin_ref[...]
    eout = dot_t(hidden, wout_ref[...]) + bout_ref[...]

    parts = []
    for b in range(BB):
        ii = lax.dot_general(a_in_ref[b], ein[b * L:(b + 1) * L, :],
                             (((1,), (0,)), ((), ())),
                             preferred_element_type=jnp.float32)
        io = lax.dot_general(a_out_ref[b], eout[b * L:(b + 1) * L, :],
                             (((1,), (0,)), ((), ())),
                             preferred_element_type=jnp.float32)
        parts.append(jnp.concatenate([ii, io], axis=1))
    gnn_in = jnp.concatenate(parts, axis=0)
    gnn_in = gnn_in + jnp.concatenate([biah_ref[...], boah_ref[...]], axis=1)

    gi = dot_t(gnn_in, wih_ref[...]) + bih_ref[...]
    gh = dot_t(hidden, whh_ref[...]) + bhh_ref[...]

    i_r, i_i, i_n = gi[:, :D], gi[:, D:2 * D], gi[:, 2 * D:]
    h_r, h_i, h_n = gh[:, :D], gh[:, D:2 * D], gh[:, 2 * D:]
    resetgate = jax.nn.sigmoid(i_r + h_r)
    inputgate = jax.nn.sigmoid(i_i + h_i)
    newgate = jnp.tanh(i_n + resetgate * h_n)
    out_ref[...] = newgate + inputgate * (hidden - newgate)


def _dense(rows, pos, a_in, a_out, lnw, lnb, win, bin_, wout, bout,
           wih, whh, bih, bhh, biah, boah, *, interpret=False):
    grid = B // BB
    fixed = lambda *shape: pl.BlockSpec(shape, lambda i: (0,) * len(shape))
    return pl.pallas_call(
        _dense_body,
        grid=(grid,),
        in_specs=[
            pl.BlockSpec((R, H), lambda i: (i, 0)),
            fixed(L, H),
            pl.BlockSpec((BB, L, L), lambda i: (i, 0, 0)),
            pl.BlockSpec((BB, L, L), lambda i: (i, 0, 0)),
            fixed(1, D), fixed(1, D),
            fixed(D, D), fixed(1, D),
            fixed(D, D), fixed(1, D),
            fixed(3 * D, 2 * D), fixed(3 * D, D),
            fixed(1, 3 * D), fixed(1, 3 * D),
            fixed(1, D), fixed(1, D),
        ],
        out_specs=pl.BlockSpec((R, D), lambda i: (i, 0)),
        out_shape=jax.ShapeDtypeStruct((ROWS, D), jnp.float32),
        interpret=interpret,
    )(rows, pos, a_in, a_out, lnw, lnb, win, bin_, wout, bout,
      wih, whh, bih, bhh, biah, boah)


def kernel(inputs, A, item_table, pos_table, ln_w, ln_b, W_in, b_in, W_out,
           b_out, w_ih, w_hh, b_ih, b_hh, b_iah, b_oah):
    idx_flat = inputs.reshape(ROWS).astype(jnp.int32)
    rows = _gather_rows(item_table, idx_flat)
    a_in, a_out = jnp.split(A, 2, axis=2)
    row2 = lambda v: v.reshape(1, -1)
    hidden = _dense(rows, pos_table[:L], a_in, a_out,
                    row2(ln_w), row2(ln_b), W_in, row2(b_in), W_out,
                    row2(b_out), w_ih, w_hh, row2(b_ih), row2(b_hh),
                    row2(b_iah), row2(b_oah))
    return hidden.reshape(B, L, D)
